# Initial kernel scaffold; baseline (speedup 1.0000x reference)
import jax
import jax.numpy as jnp
from jax.experimental import pallas as pl


def kernel(inp_features, neighbors_index, neighbors_kernel_index, neighbors_row_splits, kernel, bias):
    raise NotImplementedError("write your pallas kernel here")


# speedup vs baseline: 12.7527x; 12.7527x over previous
import jax
import jax.numpy as jnp
from jax import lax
from jax.experimental import pallas as pl
from jax.experimental.pallas import tpu as pltpu
from jax.experimental.pallas import tpu_sc as plsc

N = 10000
E = 320000
C = 128
F = 128
K = 9

NW = 32
CHUNK = 64
NCHUNK = -(-N // CHUNK)
ITERS = -(-NCHUNK // NW)
BROWS = CHUNK * K
REG = BROWS + 8
EB = 128
SPLITS_STAGE = 80
SPAD = (NCHUNK - 1) * CHUNK + SPLITS_STAGE
EPAD = E + 2 * EB
RPAD = NCHUNK * CHUNK
NZ = -(-(BROWS + 1) // EB)


def _sget(ref, idx):
    base = (idx // 16) * 16
    v = ref[pl.ds(base, 16)]
    return v[idx - base]


def _sc_body(feat_hbm, nidx_hbm, kidx_hbm, splits_hbm, zero_hbm, s_hbm,
             bucket, splits_v, nidx0, nidx1, kidx_v, cidx0, cidx1, zidx,
             feat0, feat1, zbuf, sem0, sem1):
    cid = lax.axis_index("c")
    sid = lax.axis_index("s")
    wid = cid * 16 + sid
    base_row = sid * REG

    pltpu.sync_copy(zero_hbm, zbuf)
    for t in range(NZ):
        for g in range(EB // 16):
            zidx[t, pl.ds(g * 16, 16)] = base_row + jnp.minimum(
                t * EB + g * 16 + lax.iota(jnp.int32, 16), BROWS)

    nidx_s = (nidx0, nidx1)
    cidx_s = (cidx0, cidx1)
    feat_s = (feat0, feat1)
    sem_s = (sem0, sem1)

    def chunk_body(i, carry):
        chunk = jnp.minimum(i * NW + wid, NCHUNK - 1)
        r0 = pl.multiple_of(chunk * CHUNK, 8)
        pltpu.sync_copy(splits_hbm.at[pl.ds(r0, SPLITS_STAGE)], splits_v)
        e0 = _sget(splits_v, 0)
        e1 = _sget(splits_v, CHUNK)
        eb0 = jnp.bitwise_and(e0, jnp.int32(-8))
        nblk = (e1 - eb0 + (EB - 1)) // EB

        for t in range(NZ):
            pltpu.sync_copy(zbuf, bucket.at[zidx.at[t]])
        pltpu.sync_copy(zbuf, bucket.at[zidx.at[NZ - 1]], add=True)
        plsc.subcore_barrier()

        def stage(b, s):
            eb = pl.multiple_of(eb0 + b * EB, 8)
            pltpu.sync_copy(nidx_hbm.at[pl.ds(eb, EB)], nidx_s[s])
            pltpu.async_copy(feat_hbm.at[nidx_s[s]], feat_s[s], sem_s[s])
            pltpu.sync_copy(kidx_hbm.at[pl.ds(eb, EB)], kidx_v)
            for g in range(EB // 16):
                eid = eb + g * 16 + lax.iota(jnp.int32, 16)
                kv = kidx_v[pl.ds(g * 16, 16)]
                lo = jnp.zeros((16,), jnp.int32)
                for step in (32, 16, 8, 4, 2, 1):
                    cand = lo + step
                    sv = plsc.load_gather(splits_v, [cand])
                    lo = jnp.where(sv <= eid, cand, lo)
                valid = (eid >= e0) & (eid < e1)
                comb = jnp.where(valid, lo * K + kv, BROWS)
                cidx_s[s][pl.ds(g * 16, 16)] = comb + base_row

        def finish(s):
            pltpu.make_async_copy(feat_hbm.at[nidx_s[s]], feat_s[s],
                                  sem_s[s]).wait()
            pltpu.sync_copy(feat_s[s], bucket.at[cidx_s[s]], add=True)

        @pl.when(nblk > 0)
        def _():
            stage(0, 0)

        def pair(p, carry2):
            b = p * 2

            @pl.when(b < nblk)
            def _():
                @pl.when(b + 1 < nblk)
                def _():
                    stage(b + 1, 1)
                finish(0)

            @pl.when(b + 1 < nblk)
            def _():
                @pl.when(b + 2 < nblk)
                def _():
                    stage(b + 2, 0)
                finish(1)

            return carry2

        lax.fori_loop(0, (nblk + 1) // 2, pair, 0)
        plsc.subcore_barrier()
        pltpu.sync_copy(bucket.at[pl.ds(base_row, BROWS)],
                        s_hbm.at[pl.ds(chunk * BROWS, BROWS)])

        return carry

    lax.fori_loop(0, ITERS, chunk_body, 0)


def _mm_body(s_ref, w_ref, b_ref, o_ref):
    o_ref[...] = jnp.dot(s_ref[...], w_ref[...],
                         preferred_element_type=jnp.float32) + b_ref[0:1, :]


def _sc_stage(inp_features, neighbors_index, neighbors_kernel_index,
              neighbors_row_splits):
    nidx_pad = jnp.zeros((EPAD,), jnp.int32).at[:E].set(neighbors_index)
    kidx_pad = jnp.zeros((EPAD,), jnp.int32).at[:E].set(neighbors_kernel_index)
    splits_pad = jnp.full((SPAD,), jnp.int32(E)).at[:N + 1].set(neighbors_row_splits)
    zeros_tile = jnp.zeros((EB, C), jnp.float32)

    mesh = plsc.VectorSubcoreMesh(core_axis_name="c", subcore_axis_name="s")
    sc_fn = pl.kernel(
        _sc_body,
        out_type=jax.ShapeDtypeStruct((NCHUNK * BROWS, C), jnp.float32),
        mesh=mesh,
        scratch_types=[
            pltpu.VMEM_SHARED((16 * REG, C), jnp.float32),
            pltpu.VMEM((SPLITS_STAGE,), jnp.int32),
            pltpu.VMEM((EB,), jnp.int32),
            pltpu.VMEM((EB,), jnp.int32),
            pltpu.VMEM((EB,), jnp.int32),
            pltpu.VMEM((EB,), jnp.int32),
            pltpu.VMEM((EB,), jnp.int32),
            pltpu.VMEM((NZ, EB), jnp.int32),
            pltpu.VMEM((EB, C), jnp.float32),
            pltpu.VMEM((EB, C), jnp.float32),
            pltpu.VMEM((EB, C), jnp.float32),
            pltpu.SemaphoreType.DMA,
            pltpu.SemaphoreType.DMA,
        ],
        compiler_params=pltpu.CompilerParams(needs_layout_passes=False),
    )
    return sc_fn(inp_features, nidx_pad, kidx_pad, splits_pad, zeros_tile)


@jax.jit
def kernel(inp_features, neighbors_index, neighbors_kernel_index,
           neighbors_row_splits, kernel, bias):
    S = _sc_stage(inp_features, neighbors_index, neighbors_kernel_index,
                  neighbors_row_splits)

    S2 = S.reshape(RPAD, K * C)
    W2 = kernel.reshape(K * C, F)
    bias_tile = jnp.tile(bias[None, :], (8, 1))

    BM = 1024
    out_pad = pl.pallas_call(
        _mm_body,
        grid=(pl.cdiv(RPAD, BM),),
        in_specs=[
            pl.BlockSpec((BM, K * C), lambda i: (i, 0)),
            pl.BlockSpec((K * C, F), lambda i: (0, 0)),
            pl.BlockSpec((8, F), lambda i: (0, 0)),
        ],
        out_specs=pl.BlockSpec((BM, F), lambda i: (i, 0)),
        out_shape=jax.ShapeDtypeStruct((RPAD, F), jnp.float32),
    )(S2, W2, bias_tile)

    return out_pad[:N]


# speedup vs baseline: 13.6139x; 1.0675x over previous
import jax
import jax.numpy as jnp
from jax import lax
from jax.experimental import pallas as pl
from jax.experimental.pallas import tpu as pltpu
from jax.experimental.pallas import tpu_sc as plsc

N = 10000
E = 320000
C = 128
F = 128
K = 9

NW = 32
CHUNK = 64
NCHUNK = -(-N // CHUNK)
ITERS = -(-NCHUNK // NW)
BROWS = CHUNK * K
REG = BROWS + 8
EB = 128
SPLITS_STAGE = 80
SPAD = (NCHUNK - 1) * CHUNK + SPLITS_STAGE
WBLK = 4
EPAD = E + 2 * WBLK * EB
RPAD = NCHUNK * CHUNK
NZ = -(-(BROWS + 1) // EB)


def _sget(ref, idx):
    base = (idx // 16) * 16
    v = ref[pl.ds(base, 16)]
    return v[idx - base]


def _sc_body(feat_hbm, nidx_hbm, kidx_hbm, splits_hbm, zero_hbm, s_hbm,
             bucket, splits_v, nidx_w, kidx_w, cidx0, cidx1, zidx,
             feat0, feat1, zbuf, sem0, sem1):
    cid = lax.axis_index("c")
    sid = lax.axis_index("s")
    wid = cid * 16 + sid
    base_row = sid * REG

    pltpu.sync_copy(zero_hbm, zbuf)
    for t in range(NZ):
        for g in range(EB // 16):
            zidx[t, pl.ds(g * 16, 16)] = base_row + jnp.minimum(
                t * EB + g * 16 + lax.iota(jnp.int32, 16), BROWS)

    cidx_s = (cidx0, cidx1)
    feat_s = (feat0, feat1)
    sem_s = (sem0, sem1)

    def chunk_body(i, carry):
        chunk = jnp.minimum(i * NW + wid, NCHUNK - 1)
        r0 = pl.multiple_of(chunk * CHUNK, 8)
        pltpu.sync_copy(splits_hbm.at[pl.ds(r0, SPLITS_STAGE)], splits_v)
        e0 = _sget(splits_v, 0)
        e1 = _sget(splits_v, CHUNK)
        eb0 = jnp.bitwise_and(e0, jnp.int32(-8))
        nblk = (e1 - eb0 + (EB - 1)) // EB
        nwin = (nblk + (WBLK - 1)) // WBLK

        for t in range(NZ):
            pltpu.sync_copy(zbuf, bucket.at[zidx.at[t]])
        pltpu.sync_copy(zbuf, bucket.at[zidx.at[NZ - 1]], add=True)
        plsc.subcore_barrier()

        def win_body(w, carry2):
            ew = pl.multiple_of(eb0 + w * (WBLK * EB), 8)
            pltpu.sync_copy(nidx_hbm.at[pl.ds(ew, WBLK * EB)], nidx_w)
            pltpu.sync_copy(kidx_hbm.at[pl.ds(ew, WBLK * EB)], kidx_w)
            b0 = w * WBLK

            def stage(j, s):
                eb = eb0 + (b0 + j) * EB
                pltpu.async_copy(feat_hbm.at[nidx_w.at[pl.ds(j * EB, EB)]],
                                 feat_s[s], sem_s[s])
                for g in range(EB // 16):
                    eid = eb + g * 16 + lax.iota(jnp.int32, 16)
                    kv = kidx_w[pl.ds(j * EB + g * 16, 16)]
                    lo = jnp.zeros((16,), jnp.int32)
                    for step in (32, 16, 8, 4, 2, 1):
                        cand = lo + step
                        sv = plsc.load_gather(splits_v, [cand])
                        lo = jnp.where(sv <= eid, cand, lo)
                    valid = (eid >= e0) & (eid < e1)
                    comb = jnp.where(valid, lo * K + kv, BROWS)
                    cidx_s[s][pl.ds(g * 16, 16)] = comb + base_row

            def finish(j, s):
                pltpu.make_async_copy(feat_hbm.at[nidx_w.at[pl.ds(j * EB, EB)]],
                                      feat_s[s], sem_s[s]).wait()
                pltpu.sync_copy(feat_s[s], bucket.at[cidx_s[s]], add=True)

            @pl.when(b0 < nblk)
            def _():
                stage(0, 0)

            for j in range(WBLK):
                @pl.when(b0 + j < nblk)
                def _(j=j):
                    if j + 1 < WBLK:
                        @pl.when(b0 + j + 1 < nblk)
                        def _():
                            stage(j + 1, (j + 1) % 2)
                    finish(j, j % 2)

            return carry2

        lax.fori_loop(0, nwin, win_body, 0)
        plsc.subcore_barrier()
        pltpu.sync_copy(bucket.at[pl.ds(base_row, BROWS)],
                        s_hbm.at[pl.ds(chunk * BROWS, BROWS)])

        return carry

    lax.fori_loop(0, ITERS, chunk_body, 0)


def _mm_body(s_ref, w_ref, b_ref, o_ref):
    o_ref[...] = jnp.dot(s_ref[...], w_ref[...],
                         preferred_element_type=jnp.float32) + b_ref[0:1, :]


def _sc_stage(inp_features, neighbors_index, neighbors_kernel_index,
              neighbors_row_splits):
    nidx_pad = jnp.zeros((EPAD,), jnp.int32).at[:E].set(neighbors_index)
    kidx_pad = jnp.zeros((EPAD,), jnp.int32).at[:E].set(neighbors_kernel_index)
    splits_pad = jnp.full((SPAD,), jnp.int32(E)).at[:N + 1].set(neighbors_row_splits)
    zeros_tile = jnp.zeros((EB, C), jnp.float32)

    mesh = plsc.VectorSubcoreMesh(core_axis_name="c", subcore_axis_name="s")
    sc_fn = pl.kernel(
        _sc_body,
        out_type=jax.ShapeDtypeStruct((NCHUNK * BROWS, C), jnp.float32),
        mesh=mesh,
        scratch_types=[
            pltpu.VMEM_SHARED((16 * REG, C), jnp.float32),
            pltpu.VMEM((SPLITS_STAGE,), jnp.int32),
            pltpu.VMEM((WBLK * EB,), jnp.int32),
            pltpu.VMEM((WBLK * EB,), jnp.int32),
            pltpu.VMEM((EB,), jnp.int32),
            pltpu.VMEM((EB,), jnp.int32),
            pltpu.VMEM((NZ, EB), jnp.int32),
            pltpu.VMEM((EB, C), jnp.float32),
            pltpu.VMEM((EB, C), jnp.float32),
            pltpu.VMEM((EB, C), jnp.float32),
            pltpu.SemaphoreType.DMA,
            pltpu.SemaphoreType.DMA,
        ],
        compiler_params=pltpu.CompilerParams(needs_layout_passes=False),
    )
    return sc_fn(inp_features, nidx_pad, kidx_pad, splits_pad, zeros_tile)


@jax.jit
def kernel(inp_features, neighbors_index, neighbors_kernel_index,
           neighbors_row_splits, kernel, bias):
    S = _sc_stage(inp_features, neighbors_index, neighbors_kernel_index,
                  neighbors_row_splits)

    S2 = S.reshape(RPAD, K * C)
    W2 = kernel.reshape(K * C, F)
    bias_tile = jnp.tile(bias[None, :], (8, 1))

    BM = 1024
    out_pad = pl.pallas_call(
        _mm_body,
        grid=(pl.cdiv(RPAD, BM),),
        in_specs=[
            pl.BlockSpec((BM, K * C), lambda i: (i, 0)),
            pl.BlockSpec((K * C, F), lambda i: (0, 0)),
            pl.BlockSpec((8, F), lambda i: (0, 0)),
        ],
        out_specs=pl.BlockSpec((BM, F), lambda i: (i, 0)),
        out_shape=jax.ShapeDtypeStruct((RPAD, F), jnp.float32),
    )(S2, W2, bias_tile)

    return out_pad[:N]


# speedup vs baseline: 14.3954x; 1.0574x over previous
import jax
import jax.numpy as jnp
from jax import lax
from jax.experimental import pallas as pl
from jax.experimental.pallas import tpu as pltpu
from jax.experimental.pallas import tpu_sc as plsc

N = 10000
E = 320000
C = 128
F = 128
K = 9

NW = 32
CHUNK = 64
NCHUNK = -(-N // CHUNK)
ITERS = -(-NCHUNK // NW)
BROWS = CHUNK * K
REG = BROWS + 8
EB = 128
SPLITS_STAGE = 80
SPAD = (NCHUNK - 1) * CHUNK + SPLITS_STAGE
WBLK = 4
EPAD = E + 2 * WBLK * EB
RPAD = NCHUNK * CHUNK
NZ = -(-(BROWS + 1) // EB)


def _sget(ref, idx):
    base = (idx // 16) * 16
    v = ref[pl.ds(base, 16)]
    return v[idx - base]


def _sc_body(feat_hbm, nidx_hbm, kidx_hbm, splits_hbm, zero_hbm, s_hbm,
             bucket, splits_v, nidx_w, kidx_w, cidx0, cidx1, zidx,
             feat0, feat1, zbuf, sem0, sem1, ssem0, ssem1):
    cid = lax.axis_index("c")
    sid = lax.axis_index("s")
    wid = cid * 16 + sid
    base_row = sid * REG

    pltpu.sync_copy(zero_hbm, zbuf)
    for t in range(NZ):
        for g in range(EB // 16):
            zidx[t, pl.ds(g * 16, 16)] = base_row + jnp.minimum(
                t * EB + g * 16 + lax.iota(jnp.int32, 16), BROWS)

    cidx_s = (cidx0, cidx1)
    feat_s = (feat0, feat1)
    sem_s = (sem0, sem1)
    ssem_s = (ssem0, ssem1)

    def chunk_body(i, carry):
        chunk = jnp.minimum(i * NW + wid, NCHUNK - 1)
        r0 = pl.multiple_of(chunk * CHUNK, 8)
        pltpu.sync_copy(splits_hbm.at[pl.ds(r0, SPLITS_STAGE)], splits_v)
        e0 = _sget(splits_v, 0)
        e1 = _sget(splits_v, CHUNK)
        eb0 = jnp.bitwise_and(e0, jnp.int32(-8))
        nblk = (e1 - eb0 + (EB - 1)) // EB
        nwin = (nblk + (WBLK - 1)) // WBLK

        for t in range(NZ):
            pltpu.sync_copy(zbuf, bucket.at[zidx.at[t]])
        pltpu.sync_copy(zbuf, bucket.at[zidx.at[NZ - 1]], add=True)
        plsc.subcore_barrier()

        def win_body(w, carry2):
            ew = pl.multiple_of(eb0 + w * (WBLK * EB), 8)
            pltpu.sync_copy(nidx_hbm.at[pl.ds(ew, WBLK * EB)], nidx_w)
            pltpu.sync_copy(kidx_hbm.at[pl.ds(ew, WBLK * EB)], kidx_w)
            b0 = w * WBLK

            def stage(j, s):
                @pl.when(b0 + j >= 2)
                def _():
                    wait_scatter(s)
                eb = eb0 + (b0 + j) * EB
                pltpu.async_copy(feat_hbm.at[nidx_w.at[pl.ds(j * EB, EB)]],
                                 feat_s[s], sem_s[s])
                for g in range(EB // 16):
                    eid = eb + g * 16 + lax.iota(jnp.int32, 16)
                    kv = kidx_w[pl.ds(j * EB + g * 16, 16)]
                    lo = jnp.zeros((16,), jnp.int32)
                    for step in (32, 16, 8, 4, 2, 1):
                        cand = lo + step
                        sv = plsc.load_gather(splits_v, [cand])
                        lo = jnp.where(sv <= eid, cand, lo)
                    valid = (eid >= e0) & (eid < e1)
                    comb = jnp.where(valid, lo * K + kv, BROWS)
                    cidx_s[s][pl.ds(g * 16, 16)] = comb + base_row

            def finish(j, s):
                pltpu.make_async_copy(feat_hbm.at[nidx_w.at[pl.ds(j * EB, EB)]],
                                      feat_s[s], sem_s[s]).wait()
                pltpu.async_copy(feat_s[s], bucket.at[cidx_s[s]], ssem_s[s],
                                 add=True)

            @pl.when(b0 < nblk)
            def _():
                stage(0, 0)

            for j in range(WBLK):
                @pl.when(b0 + j < nblk)
                def _(j=j):
                    if j + 1 < WBLK:
                        @pl.when(b0 + j + 1 < nblk)
                        def _():
                            stage(j + 1, (j + 1) % 2)
                    finish(j, j % 2)

            return carry2

        def wait_scatter(s):
            pltpu.make_async_copy(feat_s[s], bucket.at[cidx_s[s]],
                                  ssem_s[s]).wait()

        lax.fori_loop(0, nwin, win_body, 0)
        for s in range(2):
            cond = ((nblk >= 1) & ((nblk - 1) % 2 == s)) | \
                   ((nblk >= 2) & ((nblk - 2) % 2 == s))

            @pl.when(cond)
            def _(s=s):
                wait_scatter(s)

        plsc.subcore_barrier()
        pltpu.sync_copy(bucket.at[pl.ds(base_row, BROWS)],
                        s_hbm.at[pl.ds(chunk * BROWS, BROWS)])

        return carry

    lax.fori_loop(0, ITERS, chunk_body, 0)


def _mm_body(s_ref, w_ref, b_ref, o_ref):
    o_ref[...] = jnp.dot(s_ref[...], w_ref[...],
                         preferred_element_type=jnp.float32) + b_ref[0:1, :]


def _sc_stage(inp_features, neighbors_index, neighbors_kernel_index,
              neighbors_row_splits):
    nidx_pad = jnp.zeros((EPAD,), jnp.int32).at[:E].set(neighbors_index)
    kidx_pad = jnp.zeros((EPAD,), jnp.int32).at[:E].set(neighbors_kernel_index)
    splits_pad = jnp.full((SPAD,), jnp.int32(E)).at[:N + 1].set(neighbors_row_splits)
    zeros_tile = jnp.zeros((EB, C), jnp.float32)

    mesh = plsc.VectorSubcoreMesh(core_axis_name="c", subcore_axis_name="s")
    sc_fn = pl.kernel(
        _sc_body,
        out_type=jax.ShapeDtypeStruct((NCHUNK * BROWS, C), jnp.float32),
        mesh=mesh,
        scratch_types=[
            pltpu.VMEM_SHARED((16 * REG, C), jnp.float32),
            pltpu.VMEM((SPLITS_STAGE,), jnp.int32),
            pltpu.VMEM((WBLK * EB,), jnp.int32),
            pltpu.VMEM((WBLK * EB,), jnp.int32),
            pltpu.VMEM((EB,), jnp.int32),
            pltpu.VMEM((EB,), jnp.int32),
            pltpu.VMEM((NZ, EB), jnp.int32),
            pltpu.VMEM((EB, C), jnp.float32),
            pltpu.VMEM((EB, C), jnp.float32),
            pltpu.VMEM((EB, C), jnp.float32),
            pltpu.SemaphoreType.DMA,
            pltpu.SemaphoreType.DMA,
            pltpu.SemaphoreType.DMA,
            pltpu.SemaphoreType.DMA,
        ],
        compiler_params=pltpu.CompilerParams(needs_layout_passes=False),
    )
    return sc_fn(inp_features, nidx_pad, kidx_pad, splits_pad, zeros_tile)


@jax.jit
def kernel(inp_features, neighbors_index, neighbors_kernel_index,
           neighbors_row_splits, kernel, bias):
    S = _sc_stage(inp_features, neighbors_index, neighbors_kernel_index,
                  neighbors_row_splits)

    S2 = S.reshape(RPAD, K * C)
    W2 = kernel.reshape(K * C, F)
    bias_tile = jnp.tile(bias[None, :], (8, 1))

    BM = 1024
    out_pad = pl.pallas_call(
        _mm_body,
        grid=(pl.cdiv(RPAD, BM),),
        in_specs=[
            pl.BlockSpec((BM, K * C), lambda i: (i, 0)),
            pl.BlockSpec((K * C, F), lambda i: (0, 0)),
            pl.BlockSpec((8, F), lambda i: (0, 0)),
        ],
        out_specs=pl.BlockSpec((BM, F), lambda i: (i, 0)),
        out_shape=jax.ShapeDtypeStruct((N, F), jnp.float32),
    )(S2, W2, bias_tile)

    return out_pad


# speedup vs baseline: 14.4253x; 1.0021x over previous
import jax
import jax.numpy as jnp
from jax import lax
from jax.experimental import pallas as pl
from jax.experimental.pallas import tpu as pltpu
from jax.experimental.pallas import tpu_sc as plsc

N = 10000
E = 320000
C = 128
F = 128
K = 9

NW = 32
CHUNK = 64
NCHUNK = -(-N // CHUNK)
ITERS = -(-NCHUNK // NW)
BROWS = CHUNK * K
REG = BROWS + 8
EB = 128
SPLITS_STAGE = 80
SPAD = (NCHUNK - 1) * CHUNK + SPLITS_STAGE
WBLK = 4
EPAD = E + 2 * WBLK * EB
RPAD = NCHUNK * CHUNK
NZ = -(-(BROWS + 1) // EB)


def _sget(ref, idx):
    base = (idx // 16) * 16
    v = ref[pl.ds(base, 16)]
    return v[idx - base]


def _sc_body(feat_hbm, nidx_hbm, kidx_hbm, splits_hbm, zero_hbm, s_hbm,
             bucket, splits_v, nidx_w, kidx_w, cidx0, cidx1, zidx,
             feat0, feat1, zbuf, sem0, sem1, ssem0, ssem1, zsem):
    cid = lax.axis_index("c")
    sid = lax.axis_index("s")
    wid = cid * 16 + sid
    base_row = sid * REG

    pltpu.sync_copy(zero_hbm, zbuf)
    for t in range(NZ):
        for g in range(EB // 16):
            zidx[t, pl.ds(g * 16, 16)] = base_row + jnp.minimum(
                t * EB + g * 16 + lax.iota(jnp.int32, 16), BROWS)

    cidx_s = (cidx0, cidx1)
    feat_s = (feat0, feat1)
    sem_s = (sem0, sem1)
    ssem_s = (ssem0, ssem1)

    def chunk_body(i, carry):
        chunk = jnp.minimum(i * NW + wid, NCHUNK - 1)
        r0 = pl.multiple_of(chunk * CHUNK, 8)
        pltpu.sync_copy(splits_hbm.at[pl.ds(r0, SPLITS_STAGE)], splits_v)
        e0 = _sget(splits_v, 0)
        e1 = _sget(splits_v, CHUNK)
        eb0 = jnp.bitwise_and(e0, jnp.int32(-8))
        nblk = (e1 - eb0 + (EB - 1)) // EB
        nwin = (nblk + (WBLK - 1)) // WBLK

        for t in range(NZ):
            pltpu.async_copy(zbuf, bucket.at[zidx.at[t]], zsem)
        for t in range(NZ):
            pltpu.make_async_copy(zbuf, bucket.at[zidx.at[t]], zsem).wait()
        pltpu.sync_copy(zbuf, bucket.at[zidx.at[NZ - 1]], add=True)
        plsc.subcore_barrier()

        def win_body(w, carry2):
            ew = pl.multiple_of(eb0 + w * (WBLK * EB), 8)
            pltpu.sync_copy(nidx_hbm.at[pl.ds(ew, WBLK * EB)], nidx_w)
            pltpu.sync_copy(kidx_hbm.at[pl.ds(ew, WBLK * EB)], kidx_w)
            b0 = w * WBLK

            def stage(j, s):
                @pl.when(b0 + j >= 2)
                def _():
                    wait_scatter(s)
                eb = eb0 + (b0 + j) * EB
                pltpu.async_copy(feat_hbm.at[nidx_w.at[pl.ds(j * EB, EB)]],
                                 feat_s[s], sem_s[s])
                for g in range(EB // 16):
                    eid = eb + g * 16 + lax.iota(jnp.int32, 16)
                    kv = kidx_w[pl.ds(j * EB + g * 16, 16)]
                    lo = jnp.zeros((16,), jnp.int32)
                    for step in (32, 16, 8, 4, 2, 1):
                        cand = lo + step
                        sv = plsc.load_gather(splits_v, [cand])
                        lo = jnp.where(sv <= eid, cand, lo)
                    valid = (eid >= e0) & (eid < e1)
                    comb = jnp.where(valid, lo * K + kv, BROWS)
                    cidx_s[s][pl.ds(g * 16, 16)] = comb + base_row

            def finish(j, s):
                pltpu.make_async_copy(feat_hbm.at[nidx_w.at[pl.ds(j * EB, EB)]],
                                      feat_s[s], sem_s[s]).wait()
                pltpu.async_copy(feat_s[s], bucket.at[cidx_s[s]], ssem_s[s],
                                 add=True)

            @pl.when(b0 < nblk)
            def _():
                stage(0, 0)

            for j in range(WBLK):
                @pl.when(b0 + j < nblk)
                def _(j=j):
                    if j + 1 < WBLK:
                        @pl.when(b0 + j + 1 < nblk)
                        def _():
                            stage(j + 1, (j + 1) % 2)
                    finish(j, j % 2)

            return carry2

        def wait_scatter(s):
            pltpu.make_async_copy(feat_s[s], bucket.at[cidx_s[s]],
                                  ssem_s[s]).wait()

        lax.fori_loop(0, nwin, win_body, 0)
        for s in range(2):
            cond = ((nblk >= 1) & ((nblk - 1) % 2 == s)) | \
                   ((nblk >= 2) & ((nblk - 2) % 2 == s))

            @pl.when(cond)
            def _(s=s):
                wait_scatter(s)

        plsc.subcore_barrier()
        pltpu.sync_copy(bucket.at[pl.ds(base_row, BROWS)],
                        s_hbm.at[pl.ds(chunk * BROWS, BROWS)])

        return carry

    lax.fori_loop(0, ITERS, chunk_body, 0)


def _mm_body(s_ref, w_ref, b_ref, o_ref):
    o_ref[...] = jnp.dot(s_ref[...], w_ref[...],
                         preferred_element_type=jnp.float32) + b_ref[0:1, :]


def _sc_stage(inp_features, neighbors_index, neighbors_kernel_index,
              neighbors_row_splits):
    nidx_pad = jnp.zeros((EPAD,), jnp.int32).at[:E].set(neighbors_index)
    kidx_pad = jnp.zeros((EPAD,), jnp.int32).at[:E].set(neighbors_kernel_index)
    splits_pad = jnp.full((SPAD,), jnp.int32(E)).at[:N + 1].set(neighbors_row_splits)
    zeros_tile = jnp.zeros((EB, C), jnp.float32)

    mesh = plsc.VectorSubcoreMesh(core_axis_name="c", subcore_axis_name="s")
    sc_fn = pl.kernel(
        _sc_body,
        out_type=jax.ShapeDtypeStruct((NCHUNK * BROWS, C), jnp.float32),
        mesh=mesh,
        scratch_types=[
            pltpu.VMEM_SHARED((16 * REG, C), jnp.float32),
            pltpu.VMEM((SPLITS_STAGE,), jnp.int32),
            pltpu.VMEM((WBLK * EB,), jnp.int32),
            pltpu.VMEM((WBLK * EB,), jnp.int32),
            pltpu.VMEM((EB,), jnp.int32),
            pltpu.VMEM((EB,), jnp.int32),
            pltpu.VMEM((NZ, EB), jnp.int32),
            pltpu.VMEM((EB, C), jnp.float32),
            pltpu.VMEM((EB, C), jnp.float32),
            pltpu.VMEM((EB, C), jnp.float32),
            pltpu.SemaphoreType.DMA,
            pltpu.SemaphoreType.DMA,
            pltpu.SemaphoreType.DMA,
            pltpu.SemaphoreType.DMA,
            pltpu.SemaphoreType.DMA,
        ],
        compiler_params=pltpu.CompilerParams(needs_layout_passes=False),
    )
    return sc_fn(inp_features, nidx_pad, kidx_pad, splits_pad, zeros_tile)


@jax.jit
def kernel(inp_features, neighbors_index, neighbors_kernel_index,
           neighbors_row_splits, kernel, bias):
    S = _sc_stage(inp_features, neighbors_index, neighbors_kernel_index,
                  neighbors_row_splits)

    S2 = S.reshape(RPAD, K * C)
    W2 = kernel.reshape(K * C, F)
    bias_tile = jnp.tile(bias[None, :], (8, 1))

    BM = 1024
    out_pad = pl.pallas_call(
        _mm_body,
        grid=(pl.cdiv(RPAD, BM),),
        in_specs=[
            pl.BlockSpec((BM, K * C), lambda i: (i, 0)),
            pl.BlockSpec((K * C, F), lambda i: (0, 0)),
            pl.BlockSpec((8, F), lambda i: (0, 0)),
        ],
        out_specs=pl.BlockSpec((BM, F), lambda i: (i, 0)),
        out_shape=jax.ShapeDtypeStruct((N, F), jnp.float32),
    )(S2, W2, bias_tile)

    return out_pad
